# baseline (speedup 1.0000x reference)
import functools

import jax
import jax.numpy as jnp
from jax.experimental import pallas as pl
from jax.experimental.pallas import tpu as pltpu


def _pick_bb(batch):
    return 512 if batch % 512 == 0 else batch


def _pick_bc(c_pad):
    for bc in (384, 256, 128):
        if c_pad % bc == 0:
            return bc
    return c_pad


def _main_body(nc, num_classes, cls_ref, wb_ref, p_ref, q_ref, loss_ref, acc_ref,
               rp_ref, m_ref, a_ref, s_ref, t_ref):
    i = pl.program_id(0)
    j = pl.program_id(1)
    w = wb_ref[0]
    b = wb_ref[1]
    bc = q_ref.shape[0]
    P = p_ref[:, 0, 0, :]
    Q = q_ref[...]

    @pl.when(j == 0)
    def _():
        pnorm = jnp.sqrt(jnp.sum(P * P, axis=1))
        rp_ref[...] = 1.0 / jnp.maximum(pnorm, 1e-8)

    qnorm = jnp.sqrt(jnp.sum(Q * Q, axis=1))
    rq = 1.0 / jnp.maximum(qnorm, 1e-8)

    S = jax.lax.dot_general(P, Q, (((1,), (1,)), ((), ())),
                            preferred_element_type=jnp.float32)
    rp = rp_ref[...]
    l = (S * rp[:, None]) * (rq * w)[None, :] + b
    cglob = j * bc + jax.lax.broadcasted_iota(jnp.int32, (1, bc), 1)
    l = jnp.where(cglob < num_classes, l, -1e30)

    K = jnp.abs(w) + jnp.abs(b)
    s_tile = jnp.sum(jnp.exp(l - K), axis=1)
    m_tile = jnp.max(l, axis=1)
    idx_tile = jnp.min(jnp.where(l == m_tile[:, None], cglob, jnp.int32(2**30)),
                       axis=1)
    clsv = cls_ref[...]
    t_tile = jnp.sum(jnp.where(cglob == clsv[:, None], l, 0.0), axis=1)

    @pl.when(j == 0)
    def _():
        m_ref[...] = m_tile
        a_ref[...] = idx_tile
        s_ref[...] = s_tile
        t_ref[...] = t_tile

    @pl.when(j > 0)
    def _():
        m_old = m_ref[...]
        upd = m_tile > m_old
        m_ref[...] = jnp.maximum(m_old, m_tile)
        a_ref[...] = jnp.where(upd, idx_tile, a_ref[...])
        s_ref[...] = s_ref[...] + s_tile
        t_ref[...] = t_ref[...] + t_tile

    @pl.when(j == nc - 1)
    def _():
        row_loss = jnp.log(s_ref[...]) + K - t_ref[...]
        part_loss = jnp.sum(row_loss)
        part_acc = jnp.sum((a_ref[...] == clsv).astype(jnp.float32))

        @pl.when(i == 0)
        def _():
            loss_ref[0, 0] = part_loss
            acc_ref[0, 0] = part_acc

        @pl.when(i > 0)
        def _():
            loss_ref[0, 0] = loss_ref[0, 0] + part_loss
            acc_ref[0, 0] = acc_ref[0, 0] + part_acc


def _fused_loss(x, cls, wb, q_eff, num_classes, interpret=False):
    batch, _, _, dim = x.shape
    c_pad = q_eff.shape[0]
    bb = _pick_bb(batch)
    bc = _pick_bc(c_pad)
    nb = batch // bb
    nc = c_pad // bc
    body = functools.partial(_main_body, nc, num_classes)
    loss_sum, acc_sum = pl.pallas_call(
        body,
        grid=(nb, nc),
        in_specs=[
            pl.BlockSpec((bb,), lambda i, j: (i,)),
            pl.BlockSpec(memory_space=pltpu.SMEM),
            pl.BlockSpec((bb, 1, 1, dim), lambda i, j: (i, 0, 0, 0)),
            pl.BlockSpec((bc, dim), lambda i, j: (j, 0)),
        ],
        out_specs=[
            pl.BlockSpec(memory_space=pltpu.SMEM),
            pl.BlockSpec(memory_space=pltpu.SMEM),
        ],
        out_shape=[
            jax.ShapeDtypeStruct((1, 1), jnp.float32),
            jax.ShapeDtypeStruct((1, 1), jnp.float32),
        ],
        scratch_shapes=[
            pltpu.VMEM((bb,), jnp.float32),
            pltpu.VMEM((bb,), jnp.float32),
            pltpu.VMEM((bb,), jnp.int32),
            pltpu.VMEM((bb,), jnp.float32),
            pltpu.VMEM((bb,), jnp.float32),
        ],
        compiler_params=pltpu.CompilerParams(
            dimension_semantics=("arbitrary", "arbitrary"),
        ),
        interpret=interpret,
    )(cls, wb, x, q_eff)
    nloss = loss_sum[0, 0] / batch
    prec1 = acc_sum[0, 0] / batch * 100.0
    return nloss, prec1


def kernel(x, epoch, classes, w, b, queue, queue_ptr):
    batch = x.shape[0]
    num_classes = queue.shape[0]
    cls = classes[0]
    iota = jnp.arange(batch, dtype=jnp.int32)
    winner = jnp.full((num_classes,), -1, jnp.int32).at[cls].max(iota)
    covered = winner >= 0
    anchors = x[:, 1, :]
    q_eff = jnp.where(covered[:, None], anchors[jnp.maximum(winner, 0)],
                      queue[:, 0, :])
    c_pad = ((num_classes + 383) // 384) * 384
    q_eff = jnp.pad(q_eff, ((0, c_pad - num_classes), (0, 0)))
    wb = jnp.stack([w.astype(jnp.float32), b.astype(jnp.float32)])
    x4 = x.reshape(batch, 2, 1, x.shape[2])
    return _fused_loss(x4, cls, wb, q_eff, num_classes)
